# baseline (speedup 1.0000x reference)
import functools

import jax
import jax.numpy as jnp
from jax import lax
from jax.experimental import pallas as pl
from jax.experimental.pallas import tpu as pltpu
from jax.experimental.pallas import tpu_sc as plsc

N = 50000
M = 16
F = 64
NBR = 16
NCRY = 256
H = 128
EPS = 1e-5

B = 400
NBLK = N // B
E_TOT = N * M

LANES = 128
NW = 32

NBR_IDX_ROWS = 6400
RPW0 = 304
RPW1 = 96
C0T = 16 * RPW0
NBR_BANK_ROWS = 2


def _sc_gather(table, idx2d, bank_rows):
    n_idx_rows, _ = idx2d.shape
    D = table.shape[1]
    R = bank_rows
    nbh0 = RPW0 // R // 2
    nbh1 = RPW1 // R // 2
    assert RPW0 % (2 * R) == 0 and RPW1 % (2 * R) == 0
    assert 16 * (RPW0 + RPW1) == n_idx_rows

    def body(tab, idx, out, idx_all, rows_v, sem_g, sem_s):
        c = lax.axis_index("c")
        s = lax.axis_index("s")
        base = jnp.where(c == 0, s * RPW0, C0T + s * RPW1)
        nbh = jnp.where(c == 0, nbh0, nbh1)

        @pl.when(c == 0)
        def _():
            pltpu.sync_copy(idx.at[pl.ds(base, RPW0)], idx_all)

        @pl.when(c == 1)
        def _():
            pltpu.sync_copy(idx.at[pl.ds(base, RPW1)],
                            idx_all.at[pl.ds(0, RPW1)])

        def fire_gather(bank, j0):
            for b in range(R):
                pltpu.async_copy(
                    tab.at[idx_all.at[j0 + b]],
                    rows_v.at[pl.ds((bank * R + b) * LANES, LANES)],
                    sem_g,
                )

        def fire_store(bank, j0):
            for b in range(R):
                pltpu.async_copy(
                    rows_v.at[pl.ds((bank * R + b) * LANES, LANES)],
                    out.at[pl.ds((base + j0 + b) * LANES, LANES)],
                    sem_s,
                )

        def wait_g(n):
            for _ in range(n):
                pltpu.make_async_copy(
                    tab.at[idx_all.at[0]],
                    rows_v.at[pl.ds(0, LANES)], sem_g).wait()

        def wait_s(n):
            for _ in range(n):
                pltpu.make_async_copy(
                    rows_v.at[pl.ds(0, LANES)],
                    out.at[pl.ds(base * LANES, LANES)], sem_s).wait()

        fire_gather(0, 0)

        def loop_body(k2, carry):
            a = 2 * k2
            wait_g(R)

            @pl.when(k2 > 0)
            def _():
                wait_s(R)
            fire_gather(1, (a + 1) * R)
            fire_store(0, a * R)
            wait_g(R)
            wait_s(R)

            @pl.when(k2 < nbh - 1)
            def _():
                fire_gather(0, (a + 2) * R)
            fire_store(1, (a + 1) * R)
            return carry

        lax.fori_loop(0, nbh, loop_body, 0)
        wait_s(R)

    fn = pl.kernel(
        body,
        out_type=jax.ShapeDtypeStruct((n_idx_rows * LANES, D), table.dtype),
        mesh=plsc.VectorSubcoreMesh(core_axis_name="c", subcore_axis_name="s"),
        scratch_types=[
            pltpu.VMEM((RPW0, LANES), jnp.int32),
            pltpu.VMEM((2 * R * LANES, D), table.dtype),
            pltpu.SemaphoreType.DMA,
            pltpu.SemaphoreType.DMA,
        ],
    )
    return fn(table, idx2d)


def _sigmoid(x):
    return 1.0 / (1.0 + jnp.exp(-x))


def _softplus(x):
    return jnp.maximum(x, 0.0) + jnp.log1p(jnp.exp(-jnp.abs(x)))


def _conv_body(x_ref, g_ref, e_ref, ws, we, bf, xo_ref, s_ref, ss_ref):
    x = x_ref[...]
    sfc = jnp.dot(x, ws[...], preferred_element_type=jnp.float32) + bf[...]
    gm = g_ref[...]
    em = e_ref[...]
    h = gm + jnp.dot(em, we[...], preferred_element_type=jnp.float32)
    h3 = h.reshape(B, M, 2 * F) + sfc[:, None, :]
    msg = jnp.sum(_sigmoid(h3[..., :F]) * _softplus(h3[..., F:]), axis=1)
    xn = x + msg
    xo_ref[...] = xn
    s_ref[...] = jnp.broadcast_to(jnp.sum(xn, axis=0, keepdims=True), (8, F))
    ss_ref[...] = jnp.broadcast_to(
        jnp.sum(xn * xn, axis=0, keepdims=True), (8, F))


def _conv_call(x, g, e, ws, we, bf):
    full = lambda s: pl.BlockSpec(s, lambda i: (0, 0))
    return pl.pallas_call(
        _conv_body,
        grid=(NBLK,),
        in_specs=[
            pl.BlockSpec((B, F), lambda i: (i, 0)),
            pl.BlockSpec((B * M, 2 * F), lambda i: (i, 0)),
            pl.BlockSpec((B * M, NBR), lambda i: (i, 0)),
            full((F, 2 * F)),
            full((NBR, 2 * F)), full((1, 2 * F)),
        ],
        out_specs=[
            pl.BlockSpec((B, F), lambda i: (i, 0)),
            pl.BlockSpec((8, F), lambda i: (i, 0)),
            pl.BlockSpec((8, F), lambda i: (i, 0)),
        ],
        out_shape=[
            jax.ShapeDtypeStruct((N, F), jnp.float32),
            jax.ShapeDtypeStruct((8 * NBLK, F), jnp.float32),
            jax.ShapeDtypeStruct((8 * NBLK, F), jnp.float32),
        ],
    )(x, g, e, ws, we, bf)


def _bn_stats(s_ref, ss_ref):
    mean = jnp.sum(s_ref[...], axis=0, keepdims=True) * (1.0 / (8 * N))
    ex2 = jnp.sum(ss_ref[...], axis=0, keepdims=True) * (1.0 / (8 * N))
    var = ex2 - mean * mean
    rstd = lax.rsqrt(var + EPS)
    return mean, rstd


def _bn_body(x_ref, s_ref, ss_ref, g_ref, b_ref, wn_ref, o_ref, t_ref):
    mean, rstd = _bn_stats(s_ref, ss_ref)
    xh = (x_ref[...] - mean) * rstd
    y = _softplus(xh * g_ref[...] + b_ref[...])
    o_ref[...] = y
    t_ref[...] = jnp.dot(y, wn_ref[...], preferred_element_type=jnp.float32)


def _bn_call(x, s, ss, g, b, wn_next):
    full = lambda shp: pl.BlockSpec(shp, lambda i: (0, 0))
    return pl.pallas_call(
        _bn_body,
        grid=(NBLK,),
        in_specs=[
            pl.BlockSpec((B, F), lambda i: (i, 0)),
            full((8 * NBLK, F)), full((8 * NBLK, F)),
            full((1, F)), full((1, F)), full((F, 2 * F)),
        ],
        out_specs=[
            pl.BlockSpec((B, F), lambda i: (i, 0)),
            pl.BlockSpec((B, 2 * F), lambda i: (i, 0)),
        ],
        out_shape=[
            jax.ShapeDtypeStruct((N, F), jnp.float32),
            jax.ShapeDtypeStruct((N, 2 * F), jnp.float32),
        ],
    )(x, s, ss, g, b, wn_next)


def _embed_body(z_ref, emb_ref, wn_ref, xo_ref, t_ref):
    z = z_ref[...].reshape(1, B)
    oht = (z == lax.broadcasted_iota(jnp.int32, (128, B), 0)
           ).astype(jnp.float32)
    x128 = lax.dot_general(oht, emb_ref[...], (((0,), (0,)), ((), ())),
                           preferred_element_type=jnp.float32,
                           precision=lax.Precision.HIGHEST)
    x0 = x128[:, :F]
    xo_ref[...] = x0
    t_ref[...] = jnp.dot(x0, wn_ref[...], preferred_element_type=jnp.float32)


def _embed_call(z3, emb_pad, wn0):
    return pl.pallas_call(
        _embed_body,
        grid=(NBLK,),
        in_specs=[
            pl.BlockSpec((1, 1, B), lambda i: (i, 0, 0)),
            pl.BlockSpec((128, 2 * F), lambda i: (0, 0)),
            pl.BlockSpec((F, 2 * F), lambda i: (0, 0)),
        ],
        out_specs=[
            pl.BlockSpec((B, F), lambda i: (i, 0)),
            pl.BlockSpec((B, 2 * F), lambda i: (i, 0)),
        ],
        out_shape=[
            jax.ShapeDtypeStruct((N, F), jnp.float32),
            jax.ShapeDtypeStruct((N, 2 * F), jnp.float32),
        ],
    )(z3, emb_pad, wn0)


def _pool_body(x_ref, s_ref, ss_ref, g_ref, b_ref, cid_ref,
               w1, b1, w2, b2, wo, bo, out_ref, acc, cnt):
    i = pl.program_id(0)

    @pl.when(i == 0)
    def _():
        acc[...] = jnp.zeros_like(acc)
        cnt[...] = jnp.zeros_like(cnt)

    mean, rstd = _bn_stats(s_ref, ss_ref)
    xh = (x_ref[...] - mean) * rstd
    y = _softplus(xh * g_ref[...] + b_ref[...])

    cid = cid_ref[...].reshape(1, B)
    oht = (cid == lax.broadcasted_iota(jnp.int32, (NCRY, B), 0)
           ).astype(jnp.float32)
    acc[...] += jnp.dot(oht, y, preferred_element_type=jnp.float32,
                        precision=lax.Precision.HIGHEST)
    cnt[...] += jnp.dot(oht, jnp.ones((B, F), jnp.float32),
                        preferred_element_type=jnp.float32,
                        precision=lax.Precision.HIGHEST)

    @pl.when(i == NBLK - 1)
    def _():
        cf = acc[...] / jnp.maximum(cnt[...], 1.0)
        h1 = _softplus(jnp.dot(cf, w1[...],
                               preferred_element_type=jnp.float32) + b1[...])
        h2 = _softplus(jnp.dot(h1, w2[...],
                               preferred_element_type=jnp.float32) + b2[...])
        out_ref[...] = jnp.dot(h2, wo[...],
                               preferred_element_type=jnp.float32) + bo[...]


def _pool_call(x, s, ss, g, b, cid3, w1, b1, w2, b2, wo, bo):
    full2 = lambda shp: pl.BlockSpec(shp, lambda i: (0, 0))
    return pl.pallas_call(
        _pool_body,
        grid=(NBLK,),
        in_specs=[
            pl.BlockSpec((B, F), lambda i: (i, 0)),
            full2((8 * NBLK, F)), full2((8 * NBLK, F)),
            full2((1, F)), full2((1, F)),
            pl.BlockSpec((1, 1, B), lambda i: (i, 0, 0)),
            full2((F, H)), full2((1, H)), full2((H, F)), full2((1, F)),
            full2((F, 1)), full2((1, 1)),
        ],
        out_specs=pl.BlockSpec((NCRY, 1), lambda i: (0, 0)),
        out_shape=jax.ShapeDtypeStruct((NCRY, 1), jnp.float32),
        scratch_shapes=[
            pltpu.VMEM((NCRY, F), jnp.float32),
            pltpu.VMEM((NCRY, F), jnp.float32),
        ],
    )(x, s, ss, g, b, cid3, w1, b1, w2, b2, wo, bo)


def kernel(atom_z, nbr_fea, nbr_idx, crystal_atom_idx, atom_emb,
           W_full_0, b_full_0, bn_g_0, bn_b_0,
           W_full_1, b_full_1, bn_g_1, bn_b_1,
           W_full_2, b_full_2, bn_g_2, bn_b_2,
           W1, b1, W2, b2, Wo, bo):
    z3 = atom_z.astype(jnp.int32).reshape(NBLK, 1, B)
    nidx = jnp.pad(nbr_idx.reshape(-1).astype(jnp.int32),
                   (0, NBR_IDX_ROWS * LANES - E_TOT)).reshape(NBR_IDX_ROWS,
                                                              LANES)
    e2 = nbr_fea.reshape(E_TOT, NBR).astype(jnp.bfloat16)
    cid3 = crystal_atom_idx.astype(jnp.int32).reshape(NBLK, 1, B)

    convs = []
    for (Wf, bf, g, b) in ((W_full_0, b_full_0, bn_g_0, bn_b_0),
                           (W_full_1, b_full_1, bn_g_1, bn_b_1),
                           (W_full_2, b_full_2, bn_g_2, bn_b_2)):
        convs.append(dict(
            ws=Wf[:F, :], wn=Wf[F:2 * F, :],
            we=Wf[2 * F:, :].astype(jnp.bfloat16), bf=bf.reshape(1, 2 * F),
            g=g.reshape(1, F), b=b.reshape(1, F),
        ))

    emb_pad = jnp.pad(atom_emb, ((0, 128 - atom_emb.shape[0]), (0, F)))
    x, xt = _embed_call(z3, emb_pad, convs[0]["wn"])

    xp = s = ss = None
    for li in range(3):
        cv = convs[li]
        gath = _sc_gather(xt, nidx, NBR_BANK_ROWS)
        xp, s, ss = _conv_call(x, gath, e2, cv["ws"], cv["we"], cv["bf"])
        if li < 2:
            x, xt = _bn_call(xp, s, ss, cv["g"], cv["b"], convs[li + 1]["wn"])

    cv = convs[2]
    return _pool_call(xp, s, ss, cv["g"], cv["b"], cid3,
                      W1, b1.reshape(1, H), W2, b2.reshape(1, F),
                      Wo, bo.reshape(1, 1))
